# baseline (speedup 1.0000x reference)
import jax
import jax.numpy as jnp
from jax.experimental import pallas as pl
from jax.experimental.pallas import tpu as pltpu

H = 32
W = 32
C = 256
B = 16
P = H * W
GRID = 50
RB = 400
IPB = 8


def _tc_body(w_ref, o_ref, rloc):
    for k in range(IPB):
        rloc[pl.ds(k * W, W), :] = w_ref[pl.ds(k * GRID, W), :]
    o_ref[...] = jnp.transpose(rloc[...], (1, 0))


def kernel(mask, pos_embed_weight):
    bsz, h, w = mask.shape
    t8 = pl.pallas_call(
        _tc_body,
        grid=(H // IPB,),
        in_specs=[pl.BlockSpec((RB, C), lambda s: (s, 0))],
        out_specs=pl.BlockSpec((C, IPB * W), lambda s: (0, s)),
        out_shape=jax.ShapeDtypeStruct((C, P), jnp.float32),
        scratch_shapes=[pltpu.VMEM((IPB * W, C), jnp.float32)],
    )(pos_embed_weight)
    return jnp.broadcast_to(t8.reshape(1, C, h, w), (bsz, C, h, w))


# speedup vs baseline: 1.0915x; 1.0915x over previous
import jax
import jax.numpy as jnp
from jax.experimental import pallas as pl
from jax.experimental.pallas import tpu as pltpu

H = 32
W = 32
C = 256
B = 16
P = H * W
GRID = 50
NBLK = 1584


def _tc_body(w_ref, o_ref, rows):
    for i in range(H):
        rows[pl.ds(i * W, W), :] = w_ref[pl.ds(i * GRID, W), :]
    o_ref[...] = jnp.transpose(rows[...], (1, 0)).reshape(C, 8, 128)


def kernel(mask, pos_embed_weight):
    bsz, h, w = mask.shape
    t8 = pl.pallas_call(
        _tc_body,
        grid=(1,),
        in_specs=[pl.BlockSpec((NBLK, C), lambda b: (0, 0))],
        out_specs=pl.BlockSpec((C, 8, 128), lambda b: (0, 0, 0)),
        out_shape=jax.ShapeDtypeStruct((C, 8, 128), jnp.float32),
        scratch_shapes=[pltpu.VMEM((P, C), jnp.float32)],
    )(pos_embed_weight)
    return jnp.broadcast_to(t8.reshape(1, C, h, w), (bsz, C, h, w))
